# baseline (speedup 1.0000x reference)
import jax
import jax.numpy as jnp
from jax import lax
from jax.experimental import pallas as pl
from jax.experimental.pallas import tpu as pltpu
from jax.experimental.pallas import tpu_sc as plsc

_D = 64
_B = 16384
_H = 50
_NC, _NS = 2, 16
_NW = _NC * _NS
_BW = _B // _NW
_BLK = 128
_NQ = _BW // _BLK


def _sc_body(idx_hbm, table_hbm, out_hbm,
             idx_v, rows_v, blk_v,
             g0, g1, g2, g3, o0, o1):
    gsems = (g0, g1, g2, g3)
    osems = (o0, o1)
    wid = lax.axis_index("s") * _NC + lax.axis_index("c")
    b0w = wid * _BW

    iota16 = lax.broadcasted_iota(jnp.int32, (16,), 0)
    mtrue = iota16 >= 0

    def stage_idx(h):
        pltpu.sync_copy(idx_hbm.at[h, pl.ds(b0w, _BW)],
                        idx_v.at[pl.ds((h % 2) * _BW, _BW)])

    def fetch_rows(h, q, slot):
        hb = (h % 2) * _BW

        @pl.loop(0, _BLK // 16, unroll=4)
        def _m_loop(m):
            vec = idx_v[pl.ds(hb + q * _BLK + 16 * m, 16)]
            for l in range(16):
                r = lax.squeeze(lax.slice(vec, (l,), (l + 1,)), (0,))
                pltpu.async_copy(
                    table_hbm.at[pl.ds(r, 1)],
                    rows_v.at[slot].at[pl.ds(16 * m + l, 1)],
                    gsems[slot])

    def rows_drain(slot):
        pltpu.make_async_copy(
            table_hbm.at[pl.ds(0, _BLK)], rows_v.at[slot], gsems[slot]).wait()

    def out_desc(h, q, ob):
        return pltpu.make_async_copy(
            blk_v.at[ob], out_hbm.at[h, :, pl.ds(b0w + q * _BLK, _BLK)],
            osems[ob])

    stage_idx(0)
    for q in range(_NQ):
        fetch_rows(0, q, q)

    @pl.loop(0, _H)
    def _h_loop(h):
        @pl.when(h < _H - 1)
        def _():
            stage_idx(h + 1)

        for q in range(_NQ):
            ob = q % 2
            rows_drain(q)

            @pl.when(4 * h + q >= 2)
            def _():
                out_desc(h, q, ob).wait()

            rows_ref = rows_v.at[q]
            blk_ref = blk_v.at[ob]

            @pl.loop(0, 8)
            def _b_loop(bgrp):
                b16 = iota16 + 16 * bgrp

                @pl.loop(0, 16, unroll=8)
                def _d_loop(d):
                    rot = (iota16 + d) & 15
                    for cg in range(4):
                        v = plsc.load_gather(
                            rows_ref, [b16, rot + 16 * cg], mask=mtrue)
                        plsc.store_scatter(
                            blk_ref, [rot + 16 * cg, b16], v, mask=mtrue)

            out_desc(h, q, ob).start()

            @pl.when(h < _H - 1)
            def _():
                fetch_rows(h + 1, q, q)

    out_desc(_H - 1, _NQ - 2, 0).wait()
    out_desc(_H - 1, _NQ - 1, 1).wait()


@jax.jit
def _embed(idx_t, table):
    mesh = plsc.VectorSubcoreMesh(
        core_axis_name="c", subcore_axis_name="s",
        num_cores=_NC, num_subcores=_NS,
    )
    f = pl.kernel(
        _sc_body,
        out_type=jax.ShapeDtypeStruct((_H, _D, _B), jnp.float32),
        mesh=mesh,
        scratch_types=[
            pltpu.VMEM((2 * _BW,), jnp.int32),
            pltpu.VMEM((_NQ, _BLK, _D), jnp.float32),
            pltpu.VMEM((2, _D, _BLK), jnp.float32),
        ] + [pltpu.SemaphoreType.DMA] * 6,
        compiler_params=pltpu.CompilerParams(
            needs_layout_passes=False, disable_bounds_checks=True),
    )
    return f(idx_t, table)


def kernel(inputs, table):
    idx_t = inputs.astype(jnp.int32).T
    out_p = _embed(idx_t, table)
    return out_p.transpose(2, 0, 1)


# speedup vs baseline: 1.0079x; 1.0079x over previous
import jax
import jax.numpy as jnp
from jax import lax
from jax.experimental import pallas as pl
from jax.experimental.pallas import tpu as pltpu
from jax.experimental.pallas import tpu_sc as plsc

_D = 64
_B = 16384
_H = 50
_NC, _NS = 2, 16
_NW = _NC * _NS
_BW = _B // _NW
_BLK = 128
_NQ = _BW // _BLK


def _sc_body(idx_hbm, table_hbm, out_hbm,
             idx_v, rows_v, blk_v,
             g0, g1, g2, g3, o0, o1):
    gsems = (g0, g1, g2, g3)
    osems = (o0, o1)
    wid = lax.axis_index("s") * _NC + lax.axis_index("c")
    b0w = wid * _BW

    iota16 = lax.broadcasted_iota(jnp.int32, (16,), 0)
    mtrue = iota16 >= 0

    def stage_idx(h):
        pltpu.sync_copy(idx_hbm.at[h, pl.ds(b0w, _BW)],
                        idx_v.at[pl.ds((h % 2) * _BW, _BW)])

    def fetch_rows(h, q, slot):
        hb = (h % 2) * _BW

        @pl.loop(0, _BLK // 16, unroll=2)
        def _m_loop(m):
            vec = idx_v[pl.ds(hb + q * _BLK + 16 * m, 16)]
            for l in range(16):
                r = lax.squeeze(lax.slice(vec, (l,), (l + 1,)), (0,))
                pltpu.async_copy(
                    table_hbm.at[pl.ds(r, 1)],
                    rows_v.at[slot].at[pl.ds(16 * m + l, 1)],
                    gsems[slot])

    def rows_drain(slot):
        pltpu.make_async_copy(
            table_hbm.at[pl.ds(0, _BLK)], rows_v.at[slot], gsems[slot]).wait()

    def out_desc(h, q, ob):
        return pltpu.make_async_copy(
            blk_v.at[ob], out_hbm.at[h, :, pl.ds(b0w + q * _BLK, _BLK)],
            osems[ob])

    stage_idx(0)
    for q in range(_NQ):
        fetch_rows(0, q, q)

    @pl.loop(0, _H)
    def _h_loop(h):
        @pl.when(h < _H - 1)
        def _():
            stage_idx(h + 1)

        for q in range(_NQ):
            ob = q % 2
            rows_drain(q)

            @pl.when(4 * h + q >= 2)
            def _():
                out_desc(h, q, ob).wait()

            rows_ref = rows_v.at[q]
            blk_ref = blk_v.at[ob]

            @pl.loop(0, 8)
            def _b_loop(bgrp):
                b16 = iota16 + 16 * bgrp

                @pl.loop(0, 16, unroll=8)
                def _d_loop(d):
                    rot = (iota16 + d) & 15
                    for cg in range(4):
                        v = plsc.load_gather(
                            rows_ref, [b16, rot + 16 * cg], mask=mtrue)
                        plsc.store_scatter(
                            blk_ref, [rot + 16 * cg, b16], v, mask=mtrue)

            out_desc(h, q, ob).start()

            @pl.when(h < _H - 1)
            def _():
                fetch_rows(h + 1, q, q)

    out_desc(_H - 1, _NQ - 2, 0).wait()
    out_desc(_H - 1, _NQ - 1, 1).wait()


@jax.jit
def _embed(idx_t, table):
    mesh = plsc.VectorSubcoreMesh(
        core_axis_name="c", subcore_axis_name="s",
        num_cores=_NC, num_subcores=_NS,
    )
    f = pl.kernel(
        _sc_body,
        out_type=jax.ShapeDtypeStruct((_H, _D, _B), jnp.float32),
        mesh=mesh,
        scratch_types=[
            pltpu.VMEM((2 * _BW,), jnp.int32),
            pltpu.VMEM((_NQ, _BLK, _D), jnp.float32),
            pltpu.VMEM((2, _D, _BLK), jnp.float32),
        ] + [pltpu.SemaphoreType.DMA] * 6,
        compiler_params=pltpu.CompilerParams(
            needs_layout_passes=False, disable_bounds_checks=True),
    )
    return f(idx_t, table)


def kernel(inputs, table):
    idx_t = inputs.astype(jnp.int32).T
    out_p = _embed(idx_t, table)
    return out_p.transpose(2, 0, 1)
